# baseline (speedup 1.0000x reference)
import functools

import jax
import jax.numpy as jnp
from jax import lax
from jax.experimental import pallas as pl
from jax.experimental.pallas import tpu as pltpu
from jax.experimental.pallas import tpu_sc as plsc

B = 16384
D = 64
NW = 32
BPW = B // NW
CH = 128
NCH = BPW // CH


def _sc_gather(u_ids, i_ids, item2ent, user_emb, item_emb, ent_emb):
    mesh = plsc.VectorSubcoreMesh(core_axis_name="c", subcore_axis_name="s")

    @functools.partial(
        pl.kernel,
        out_type=(
            jax.ShapeDtypeStruct((B, D), jnp.float32),
            jax.ShapeDtypeStruct((B, D), jnp.float32),
            jax.ShapeDtypeStruct((B, D), jnp.float32),
        ),
        mesh=mesh,
        compiler_params=pltpu.CompilerParams(use_tc_tiling_on_sc=False),
        scratch_types=[
            pltpu.VMEM((NCH, CH), jnp.int32),
            pltpu.VMEM((NCH, CH), jnp.int32),
            pltpu.VMEM((NCH, CH), jnp.int32),
            pltpu.VMEM((BPW, D), jnp.float32),
            pltpu.VMEM((BPW, D), jnp.float32),
            pltpu.VMEM((BPW, D), jnp.float32),
            pltpu.SemaphoreType.DMA,
        ],
    )
    def sc(u_ids_h, i_ids_h, i2e_h, uemb_h, iemb_h, eemb_h,
           u_out, i_out, e_out, uidx, iidx, evar, urows, irows, erows, sem):
        wid = lax.axis_index("s") * 2 + lax.axis_index("c")
        base = wid * BPW
        for k in range(NCH):
            pltpu.sync_copy(u_ids_h.at[pl.ds(base + k * CH, CH)], uidx.at[k])
            pltpu.sync_copy(i_ids_h.at[pl.ds(base + k * CH, CH)], iidx.at[k])
        lvl1 = [pltpu.async_copy(i2e_h.at[iidx.at[k]], evar.at[k], sem)
                for k in range(NCH)]
        for c in lvl1:
            c.wait()
        copies = []
        for k in range(NCH):
            sl = pl.ds(k * CH, CH)
            copies.append(pltpu.async_copy(uemb_h.at[uidx.at[k]], urows.at[sl], sem))
            copies.append(pltpu.async_copy(iemb_h.at[iidx.at[k]], irows.at[sl], sem))
            copies.append(pltpu.async_copy(eemb_h.at[evar.at[k]], erows.at[sl], sem))
        for c in copies:
            c.wait()
        pltpu.sync_copy(urows, u_out.at[pl.ds(base, BPW)])
        pltpu.sync_copy(irows, i_out.at[pl.ds(base, BPW)])
        pltpu.sync_copy(erows, e_out.at[pl.ds(base, BPW)])

    return sc(u_ids, i_ids, item2ent, user_emb, item_emb, ent_emb)


def _tc_dense(u_rows, i_rows, e_rows, pref_emb, pref_norm_emb, rel_emb, norm_emb):
    BLK = 1024

    def body(u_ref, i_ref, e_ref, pref_ref, pn_ref, rel_ref, nm_ref, out_ref):
        u = u_ref[...]
        ie = i_ref[...] + e_ref[...]
        x = u + ie
        dvec = u - ie
        pr = pref_ref[...] + rel_ref[...]
        pn = pn_ref[...] + nm_ref[...]
        pre = lax.dot_general(x, pr, (((1,), (1,)), ((), ())),
                              preferred_element_type=jnp.float32) * 0.5
        r_e = lax.dot_general(pre, pr, (((1,), (0,)), ((), ())),
                              preferred_element_type=jnp.float32) * 0.5
        nrm = lax.dot_general(pre, pn, (((1,), (0,)), ((), ())),
                              preferred_element_type=jnp.float32) * 0.5
        t = jnp.sum(dvec * nrm, axis=-1, keepdims=True)
        out_ref[...] = jnp.sum(jnp.abs(dvec + r_e - t * nrm), axis=-1)

    return pl.pallas_call(
        body,
        grid=(B // BLK,),
        in_specs=[
            pl.BlockSpec((BLK, D), lambda i: (i, 0)),
            pl.BlockSpec((BLK, D), lambda i: (i, 0)),
            pl.BlockSpec((BLK, D), lambda i: (i, 0)),
            pl.BlockSpec((64, 64), lambda i: (0, 0)),
            pl.BlockSpec((64, 64), lambda i: (0, 0)),
            pl.BlockSpec((64, 64), lambda i: (0, 0)),
            pl.BlockSpec((64, 64), lambda i: (0, 0)),
        ],
        out_specs=pl.BlockSpec((BLK,), lambda i: (i,)),
        out_shape=jax.ShapeDtypeStruct((B,), jnp.float32),
    )(u_rows, i_rows, e_rows, pref_emb, pref_norm_emb, rel_emb, norm_emb)


def kernel(u_ids, i_ids, item2ent, user_emb, item_emb, ent_emb,
           pref_emb, pref_norm_emb, rel_emb, norm_emb):
    u_rows, i_rows, e_rows = _sc_gather(u_ids, i_ids, item2ent,
                                        user_emb, item_emb, ent_emb)
    return _tc_dense(u_rows, i_rows, e_rows,
                     pref_emb, pref_norm_emb, rel_emb, norm_emb)


# speedup vs baseline: 1.2563x; 1.2563x over previous
import functools

import jax
import jax.numpy as jnp
from jax import lax
from jax.experimental import pallas as pl
from jax.experimental.pallas import tpu as pltpu
from jax.experimental.pallas import tpu_sc as plsc

B = 16384
D = 64
NW = 32
BPW = B // NW
CH = 128
NCH = BPW // CH
G = 16


def _sc_gather(u_ids, i_ids, item2ent, user_emb, item_emb, ent_emb):
    mesh = plsc.VectorSubcoreMesh(core_axis_name="c", subcore_axis_name="s")

    @functools.partial(
        pl.kernel,
        out_type=(
            jax.ShapeDtypeStruct((B, D), jnp.float32),
            jax.ShapeDtypeStruct((B, D), jnp.float32),
            jax.ShapeDtypeStruct((B, D), jnp.float32),
        ),
        mesh=mesh,
        scratch_types=[
            pltpu.VMEM((BPW,), jnp.int32),
            pltpu.VMEM((BPW,), jnp.int32),
            pltpu.VMEM((NCH, CH), jnp.int32),
            pltpu.VMEM((BPW,), jnp.int32),
            pltpu.VMEM((BPW // 2, D), jnp.float32),
            pltpu.VMEM((BPW // 2, D), jnp.float32),
            pltpu.VMEM((BPW // 2, D), jnp.float32),
            pltpu.SemaphoreType.DMA,
            pltpu.SemaphoreType.DMA,
        ],
    )
    def sc(u_ids_h, i_ids_h, i2e_h, uemb_h, iemb_h, eemb_h,
           u_out, i_out, e_out, uidx, iidx, iidx2, evar, ubuf, ibuf, ebuf,
           sem, sem2):
        wid = lax.axis_index("s") * 2 + lax.axis_index("c")
        base = wid * BPW
        pltpu.sync_copy(u_ids_h.at[pl.ds(base, BPW)], uidx)
        pltpu.sync_copy(i_ids_h.at[pl.ds(base, BPW)], iidx)
        for k in range(NCH):
            pltpu.sync_copy(i_ids_h.at[pl.ds(base + k * CH, CH)], iidx2.at[k])
        lvl1 = [pltpu.async_copy(i2e_h.at[iidx2.at[k]],
                                 evar.at[pl.ds(k * CH, CH)], sem2)
                for k in range(NCH)]
        for c in lvl1:
            c.wait()

        HB = BPW // 2
        for h in range(2):

            def fetch_group(it, carry, h=h):
                pos = h * HB + it * G
                uvec = uidx[pl.ds(pos, G)]
                ivec = iidx[pl.ds(pos, G)]
                evec = evar[pl.ds(pos, G)]
                copies = []
                for g in range(G):
                    ru = uvec[g]
                    ri = ivec[g]
                    re = evec[g]
                    dst = pl.ds(it * G + g, 1)
                    copies.append(pltpu.async_copy(
                        uemb_h.at[pl.ds(ru, 1)], ubuf.at[dst], sem))
                    copies.append(pltpu.async_copy(
                        iemb_h.at[pl.ds(ri, 1)], ibuf.at[dst], sem))
                    copies.append(pltpu.async_copy(
                        eemb_h.at[pl.ds(re, 1)], ebuf.at[dst], sem))
                for c in copies:
                    c.wait()
                return carry

            lax.fori_loop(0, HB // G, fetch_group, 0)

            out_sl = pl.ds(base + h * HB, HB)
            pltpu.sync_copy(ubuf, u_out.at[out_sl])
            pltpu.sync_copy(ibuf, i_out.at[out_sl])
            pltpu.sync_copy(ebuf, e_out.at[out_sl])

    return sc(u_ids, i_ids, item2ent, user_emb, item_emb, ent_emb)


def _tc_dense(u_rows, i_rows, e_rows, pref_emb, pref_norm_emb, rel_emb, norm_emb):
    BLK = 1024

    def body(u_ref, i_ref, e_ref, pref_ref, pn_ref, rel_ref, nm_ref, out_ref):
        u = u_ref[...]
        ie = i_ref[...] + e_ref[...]
        x = u + ie
        dvec = u - ie
        pr = pref_ref[...] + rel_ref[...]
        pn = pn_ref[...] + nm_ref[...]
        pre = lax.dot_general(x, pr, (((1,), (1,)), ((), ())),
                              preferred_element_type=jnp.float32) * 0.5
        r_e = lax.dot_general(pre, pr, (((1,), (0,)), ((), ())),
                              preferred_element_type=jnp.float32) * 0.5
        nrm = lax.dot_general(pre, pn, (((1,), (0,)), ((), ())),
                              preferred_element_type=jnp.float32) * 0.5
        t = jnp.sum(dvec * nrm, axis=-1, keepdims=True)
        out_ref[...] = jnp.sum(jnp.abs(dvec + r_e - t * nrm), axis=-1)

    return pl.pallas_call(
        body,
        grid=(B // BLK,),
        in_specs=[
            pl.BlockSpec((BLK, D), lambda i: (i, 0)),
            pl.BlockSpec((BLK, D), lambda i: (i, 0)),
            pl.BlockSpec((BLK, D), lambda i: (i, 0)),
            pl.BlockSpec((64, 64), lambda i: (0, 0)),
            pl.BlockSpec((64, 64), lambda i: (0, 0)),
            pl.BlockSpec((64, 64), lambda i: (0, 0)),
            pl.BlockSpec((64, 64), lambda i: (0, 0)),
        ],
        out_specs=pl.BlockSpec((BLK,), lambda i: (i,)),
        out_shape=jax.ShapeDtypeStruct((B,), jnp.float32),
    )(u_rows, i_rows, e_rows, pref_emb, pref_norm_emb, rel_emb, norm_emb)


def kernel(u_ids, i_ids, item2ent, user_emb, item_emb, ent_emb,
           pref_emb, pref_norm_emb, rel_emb, norm_emb):
    u_rows, i_rows, e_rows = _sc_gather(u_ids, i_ids, item2ent,
                                        user_emb, item_emb, ent_emb)
    return _tc_dense(u_rows, i_rows, e_rows,
                     pref_emb, pref_norm_emb, rel_emb, norm_emb)


# speedup vs baseline: 1.2743x; 1.0144x over previous
import functools

import jax
import jax.numpy as jnp
from jax import lax
from jax.experimental import pallas as pl
from jax.experimental.pallas import tpu as pltpu
from jax.experimental.pallas import tpu_sc as plsc

B = 16384
D = 64
NW = 32
BPW = B // NW
CH = 128
NCH = BPW // CH
G = 16


def _sc_gather(u_ids, i_ids, item2ent, user_emb, item_emb, ent_emb):
    mesh = plsc.VectorSubcoreMesh(core_axis_name="c", subcore_axis_name="s")

    @functools.partial(
        pl.kernel,
        out_type=(
            jax.ShapeDtypeStruct((B, D), jnp.float32),
            jax.ShapeDtypeStruct((B, D), jnp.float32),
            jax.ShapeDtypeStruct((B, D), jnp.float32),
        ),
        mesh=mesh,
        scratch_types=[
            pltpu.VMEM((BPW,), jnp.int32),
            pltpu.VMEM((BPW,), jnp.int32),
            pltpu.VMEM((NCH, CH), jnp.int32),
            pltpu.VMEM((BPW,), jnp.int32),
            pltpu.VMEM((BPW // 2, D), jnp.float32),
            pltpu.VMEM((BPW // 2, D), jnp.float32),
            pltpu.VMEM((BPW // 2, D), jnp.float32),
            pltpu.SemaphoreType.DMA,
            pltpu.SemaphoreType.DMA,
        ],
    )
    def sc(u_ids_h, i_ids_h, i2e_h, uemb_h, iemb_h, eemb_h,
           u_out, i_out, e_out, uidx, iidx, iidx2, evar, ubuf, ibuf, ebuf,
           sem, sem2):
        wid = lax.axis_index("s") * 2 + lax.axis_index("c")
        base = wid * BPW
        pltpu.sync_copy(u_ids_h.at[pl.ds(base, BPW)], uidx)
        pltpu.sync_copy(i_ids_h.at[pl.ds(base, BPW)], iidx)
        for k in range(NCH):
            pltpu.sync_copy(i_ids_h.at[pl.ds(base + k * CH, CH)], iidx2.at[k])
        lvl1 = [pltpu.async_copy(i2e_h.at[iidx2.at[k]],
                                 evar.at[pl.ds(k * CH, CH)], sem2)
                for k in range(NCH)]
        for c in lvl1:
            c.wait()

        HB = BPW // 2
        for h in range(2):

            def fetch_group(it, carry, h=h):
                pos = h * HB + it * G
                uvec = uidx[pl.ds(pos, G)]
                ivec = iidx[pl.ds(pos, G)]
                evec = evar[pl.ds(pos, G)]
                copies = []
                for g in range(G):
                    ru = uvec[g]
                    ri = ivec[g]
                    re = evec[g]
                    dst = pl.ds(it * G + g, 1)
                    copies.append(pltpu.async_copy(
                        uemb_h.at[pl.ds(ru, 1)], ubuf.at[dst], sem))
                    copies.append(pltpu.async_copy(
                        iemb_h.at[pl.ds(ri, 1)], ibuf.at[dst], sem))
                    copies.append(pltpu.async_copy(
                        eemb_h.at[pl.ds(re, 1)], ebuf.at[dst], sem))
                for c in copies:
                    c.wait()
                return carry

            lax.fori_loop(0, HB // G, fetch_group, 0)

            out_sl = pl.ds(base + h * HB, HB)
            pltpu.sync_copy(ubuf, u_out.at[out_sl])
            pltpu.sync_copy(ibuf, i_out.at[out_sl])
            pltpu.sync_copy(ebuf, e_out.at[out_sl])

    return sc(u_ids, i_ids, item2ent, user_emb, item_emb, ent_emb)


def _tc_dense(u_rows, i_rows, e_rows, pref_emb, pref_norm_emb, rel_emb, norm_emb):
    BLK = 1024

    def body(u_ref, i_ref, e_ref, pref_ref, pn_ref, rel_ref, nm_ref, out_ref):
        u = u_ref[...]
        ie = i_ref[...] + e_ref[...]
        x = u + ie
        dvec = u - ie
        pr = pref_ref[...] + rel_ref[...]
        pn = pn_ref[...] + nm_ref[...]
        pre = lax.dot_general(x, pr, (((1,), (1,)), ((), ())),
                              preferred_element_type=jnp.float32) * 0.5
        r_e = lax.dot_general(pre, pr, (((1,), (0,)), ((), ())),
                              preferred_element_type=jnp.float32) * 0.5
        nrm = lax.dot_general(pre, pn, (((1,), (0,)), ((), ())),
                              preferred_element_type=jnp.float32) * 0.5
        ones = jnp.ones((D, 1), jnp.float32)
        t = lax.dot_general(dvec * nrm, ones, (((1,), (0,)), ((), ())),
                            preferred_element_type=jnp.float32)
        s = lax.dot_general(jnp.abs(dvec + r_e - t * nrm), ones,
                            (((1,), (0,)), ((), ())),
                            preferred_element_type=jnp.float32)
        out_ref[...] = s[:, 0]

    return pl.pallas_call(
        body,
        grid=(B // BLK,),
        in_specs=[
            pl.BlockSpec((BLK, D), lambda i: (i, 0)),
            pl.BlockSpec((BLK, D), lambda i: (i, 0)),
            pl.BlockSpec((BLK, D), lambda i: (i, 0)),
            pl.BlockSpec((64, 64), lambda i: (0, 0)),
            pl.BlockSpec((64, 64), lambda i: (0, 0)),
            pl.BlockSpec((64, 64), lambda i: (0, 0)),
            pl.BlockSpec((64, 64), lambda i: (0, 0)),
        ],
        out_specs=pl.BlockSpec((BLK,), lambda i: (i,)),
        out_shape=jax.ShapeDtypeStruct((B,), jnp.float32),
    )(u_rows, i_rows, e_rows, pref_emb, pref_norm_emb, rel_emb, norm_emb)


def kernel(u_ids, i_ids, item2ent, user_emb, item_emb, ent_emb,
           pref_emb, pref_norm_emb, rel_emb, norm_emb):
    u_rows, i_rows, e_rows = _sc_gather(u_ids, i_ids, item2ent,
                                        user_emb, item_emb, ent_emb)
    return _tc_dense(u_rows, i_rows, e_rows,
                     pref_emb, pref_norm_emb, rel_emb, norm_emb)


# speedup vs baseline: 1.3365x; 1.0489x over previous
import functools

import jax
import jax.numpy as jnp
from jax import lax
from jax.experimental import pallas as pl
from jax.experimental.pallas import tpu as pltpu
from jax.experimental.pallas import tpu_sc as plsc

B = 16384
D = 64
NW = 32
BPW = B // NW
CH = 128
NCH = BPW // CH
G = 16
HB = BPW // 2

_MESH = dict(core_axis_name="c", subcore_axis_name="s")


def _worker_base():
    return (lax.axis_index("s") * 2 + lax.axis_index("c")) * BPW


def _ent_lookup(i_ids, item2ent):
    @functools.partial(
        pl.kernel,
        out_type=jax.ShapeDtypeStruct((B,), jnp.int32),
        mesh=plsc.VectorSubcoreMesh(**_MESH),
        scratch_types=[
            pltpu.VMEM((NCH, CH), jnp.int32),
            pltpu.VMEM((BPW,), jnp.int32),
            pltpu.SemaphoreType.DMA,
        ],
    )
    def sc(i_ids_h, i2e_h, ev_out, iidx2, evbuf, sem):
        base = _worker_base()
        for k in range(NCH):
            pltpu.sync_copy(i_ids_h.at[pl.ds(base + k * CH, CH)], iidx2.at[k])
        copies = [pltpu.async_copy(i2e_h.at[iidx2.at[k]],
                                   evbuf.at[pl.ds(k * CH, CH)], sem)
                  for k in range(NCH)]
        for c in copies:
            c.wait()
        pltpu.sync_copy(evbuf, ev_out.at[pl.ds(base, BPW)])

    return sc(i_ids, item2ent)


def _row_gather(ids, table):
    @functools.partial(
        pl.kernel,
        out_type=jax.ShapeDtypeStruct((B, D), jnp.float32),
        mesh=plsc.VectorSubcoreMesh(**_MESH),
        scratch_types=[
            pltpu.VMEM((BPW,), jnp.int32),
            pltpu.VMEM((HB, D), jnp.float32),
            pltpu.SemaphoreType.DMA,
        ],
    )
    def sc(ids_h, table_h, rows_out, idx, buf, sem):
        base = _worker_base()
        pltpu.sync_copy(ids_h.at[pl.ds(base, BPW)], idx)
        for h in range(2):

            def fetch_group(it, carry, h=h):
                pos = h * HB + it * G
                vec = idx[pl.ds(pos, G)]
                copies = []
                for g in range(G):
                    r = vec[g]
                    copies.append(pltpu.async_copy(
                        table_h.at[pl.ds(r, 1)],
                        buf.at[pl.ds(it * G + g, 1)], sem))
                for c in copies:
                    c.wait()
                return carry

            lax.fori_loop(0, HB // G, fetch_group, 0)
            pltpu.sync_copy(buf, rows_out.at[pl.ds(base + h * HB, HB)])

    return sc(ids, table)


def _tc_dense(u_rows, i_rows, e_rows, pref_emb, pref_norm_emb, rel_emb, norm_emb):
    BLK = 2048

    def body(u_ref, i_ref, e_ref, pref_ref, pn_ref, rel_ref, nm_ref, out_ref):
        u = u_ref[...]
        ie = i_ref[...] + e_ref[...]
        x = u + ie
        dvec = u - ie
        pr = pref_ref[...] + rel_ref[...]
        pn = pn_ref[...] + nm_ref[...]
        pre = lax.dot_general(x, pr, (((1,), (1,)), ((), ())),
                              preferred_element_type=jnp.float32) * 0.5
        r_e = lax.dot_general(pre, pr, (((1,), (0,)), ((), ())),
                              preferred_element_type=jnp.float32) * 0.5
        nrm = lax.dot_general(pre, pn, (((1,), (0,)), ((), ())),
                              preferred_element_type=jnp.float32) * 0.5
        ones = jnp.ones((D, 1), jnp.float32)
        t = lax.dot_general(dvec * nrm, ones, (((1,), (0,)), ((), ())),
                            preferred_element_type=jnp.float32)
        s = lax.dot_general(jnp.abs(dvec + r_e - t * nrm), ones,
                            (((1,), (0,)), ((), ())),
                            preferred_element_type=jnp.float32)
        out_ref[...] = s[:, 0]

    return pl.pallas_call(
        body,
        grid=(B // BLK,),
        in_specs=[
            pl.BlockSpec((BLK, D), lambda i: (i, 0)),
            pl.BlockSpec((BLK, D), lambda i: (i, 0)),
            pl.BlockSpec((BLK, D), lambda i: (i, 0)),
            pl.BlockSpec((64, 64), lambda i: (0, 0)),
            pl.BlockSpec((64, 64), lambda i: (0, 0)),
            pl.BlockSpec((64, 64), lambda i: (0, 0)),
            pl.BlockSpec((64, 64), lambda i: (0, 0)),
        ],
        out_specs=pl.BlockSpec((BLK,), lambda i: (i,)),
        out_shape=jax.ShapeDtypeStruct((B,), jnp.float32),
    )(u_rows, i_rows, e_rows, pref_emb, pref_norm_emb, rel_emb, norm_emb)


def kernel(u_ids, i_ids, item2ent, user_emb, item_emb, ent_emb,
           pref_emb, pref_norm_emb, rel_emb, norm_emb):
    e_var = _ent_lookup(i_ids, item2ent)
    e_rows = _row_gather(e_var, ent_emb)
    u_rows = _row_gather(u_ids, user_emb)
    i_rows = _row_gather(i_ids, item_emb)
    return _tc_dense(u_rows, i_rows, e_rows,
                     pref_emb, pref_norm_emb, rel_emb, norm_emb)


# speedup vs baseline: 1.4936x; 1.1175x over previous
import functools

import jax
import jax.numpy as jnp
from jax import lax
from jax.experimental import pallas as pl
from jax.experimental.pallas import tpu as pltpu
from jax.experimental.pallas import tpu_sc as plsc

B = 16384
D = 64
NW = 32
BPW = B // NW
CH = 128
NCH = BPW // CH
G = 16
HB = BPW // 2

_MESH = dict(core_axis_name="c", subcore_axis_name="s")


def _worker_base():
    return (lax.axis_index("s") * 2 + lax.axis_index("c")) * BPW


def _ent_lookup(i_ids, item2ent):
    @functools.partial(
        pl.kernel,
        out_type=jax.ShapeDtypeStruct((B,), jnp.int32),
        mesh=plsc.VectorSubcoreMesh(**_MESH),
        scratch_types=[
            pltpu.VMEM((NCH, CH), jnp.int32),
            pltpu.VMEM((BPW,), jnp.int32),
            pltpu.SemaphoreType.DMA,
        ],
    )
    def sc(i_ids_h, i2e_h, ev_out, iidx2, evbuf, sem):
        base = _worker_base()
        for k in range(NCH):
            pltpu.sync_copy(i_ids_h.at[pl.ds(base + k * CH, CH)], iidx2.at[k])
        copies = [pltpu.async_copy(i2e_h.at[iidx2.at[k]],
                                   evbuf.at[pl.ds(k * CH, CH)], sem)
                  for k in range(NCH)]
        for c in copies:
            c.wait()
        pltpu.sync_copy(evbuf, ev_out.at[pl.ds(base, BPW)])

    return sc(i_ids, item2ent)


def _row_gather(ids, table):
    @functools.partial(
        pl.kernel,
        out_type=jax.ShapeDtypeStruct((B, D), jnp.float32),
        mesh=plsc.VectorSubcoreMesh(**_MESH),
        scratch_types=[
            pltpu.VMEM((BPW,), jnp.int32),
            pltpu.VMEM((HB, D), jnp.float32),
            pltpu.SemaphoreType.DMA,
        ],
    )
    def sc(ids_h, table_h, rows_out, idx, buf, sem):
        base = _worker_base()
        pltpu.sync_copy(ids_h.at[pl.ds(base, BPW)], idx)

        def drain_group():
            for _ in range(G):
                pltpu.make_async_copy(table_h.at[pl.ds(0, 1)],
                                      buf.at[pl.ds(0, 1)], sem).wait()

        for h in range(2):

            def fetch_group(it, carry, h=h):
                pos = h * HB + it * G
                vec = idx[pl.ds(pos, G)]
                for g in range(G):
                    r = vec[g]
                    pltpu.async_copy(table_h.at[pl.ds(r, 1)],
                                     buf.at[pl.ds(it * G + g, 1)], sem)

                @pl.when(it != 0)
                def _():
                    drain_group()

                return carry

            lax.fori_loop(0, HB // G, fetch_group, 0)
            drain_group()
            pltpu.sync_copy(buf, rows_out.at[pl.ds(base + h * HB, HB)])

    return sc(ids, table)


def _tc_dense(u_rows, i_rows, e_rows, pref_emb, pref_norm_emb, rel_emb, norm_emb):
    BLK = 2048

    def body(u_ref, i_ref, e_ref, pref_ref, pn_ref, rel_ref, nm_ref, out_ref):
        u = u_ref[...]
        ie = i_ref[...] + e_ref[...]
        x = u + ie
        dvec = u - ie
        pr = pref_ref[...] + rel_ref[...]
        pn = pn_ref[...] + nm_ref[...]
        pre = lax.dot_general(x, pr, (((1,), (1,)), ((), ())),
                              preferred_element_type=jnp.float32) * 0.5
        r_e = lax.dot_general(pre, pr, (((1,), (0,)), ((), ())),
                              preferred_element_type=jnp.float32) * 0.5
        nrm = lax.dot_general(pre, pn, (((1,), (0,)), ((), ())),
                              preferred_element_type=jnp.float32) * 0.5
        ones = jnp.ones((D, 1), jnp.float32)
        t = lax.dot_general(dvec * nrm, ones, (((1,), (0,)), ((), ())),
                            preferred_element_type=jnp.float32)
        s = lax.dot_general(jnp.abs(dvec + r_e - t * nrm), ones,
                            (((1,), (0,)), ((), ())),
                            preferred_element_type=jnp.float32)
        out_ref[...] = s[:, 0]

    return pl.pallas_call(
        body,
        grid=(B // BLK,),
        in_specs=[
            pl.BlockSpec((BLK, D), lambda i: (i, 0)),
            pl.BlockSpec((BLK, D), lambda i: (i, 0)),
            pl.BlockSpec((BLK, D), lambda i: (i, 0)),
            pl.BlockSpec((64, 64), lambda i: (0, 0)),
            pl.BlockSpec((64, 64), lambda i: (0, 0)),
            pl.BlockSpec((64, 64), lambda i: (0, 0)),
            pl.BlockSpec((64, 64), lambda i: (0, 0)),
        ],
        out_specs=pl.BlockSpec((BLK,), lambda i: (i,)),
        out_shape=jax.ShapeDtypeStruct((B,), jnp.float32),
    )(u_rows, i_rows, e_rows, pref_emb, pref_norm_emb, rel_emb, norm_emb)


def kernel(u_ids, i_ids, item2ent, user_emb, item_emb, ent_emb,
           pref_emb, pref_norm_emb, rel_emb, norm_emb):
    e_var = _ent_lookup(i_ids, item2ent)
    e_rows = _row_gather(e_var, ent_emb)
    u_rows = _row_gather(u_ids, user_emb)
    i_rows = _row_gather(i_ids, item_emb)
    return _tc_dense(u_rows, i_rows, e_rows,
                     pref_emb, pref_norm_emb, rel_emb, norm_emb)


# speedup vs baseline: 1.5115x; 1.0120x over previous
import functools

import jax
import jax.numpy as jnp
from jax import lax
from jax.experimental import pallas as pl
from jax.experimental.pallas import tpu as pltpu
from jax.experimental.pallas import tpu_sc as plsc

B = 16384
D = 64
NW = 32
BPW = B // NW
CH = 128
NCH = BPW // CH
G = 32
HB = BPW // 2

_MESH = dict(core_axis_name="c", subcore_axis_name="s")


def _worker_base():
    return (lax.axis_index("s") * 2 + lax.axis_index("c")) * BPW


def _ent_lookup(i_ids, item2ent):
    @functools.partial(
        pl.kernel,
        out_type=jax.ShapeDtypeStruct((B,), jnp.int32),
        mesh=plsc.VectorSubcoreMesh(**_MESH),
        scratch_types=[
            pltpu.VMEM((NCH, CH), jnp.int32),
            pltpu.VMEM((BPW,), jnp.int32),
            pltpu.SemaphoreType.DMA,
        ],
    )
    def sc(i_ids_h, i2e_h, ev_out, iidx2, evbuf, sem):
        base = _worker_base()
        for k in range(NCH):
            pltpu.sync_copy(i_ids_h.at[pl.ds(base + k * CH, CH)], iidx2.at[k])
        copies = [pltpu.async_copy(i2e_h.at[iidx2.at[k]],
                                   evbuf.at[pl.ds(k * CH, CH)], sem)
                  for k in range(NCH)]
        for c in copies:
            c.wait()
        pltpu.sync_copy(evbuf, ev_out.at[pl.ds(base, BPW)])

    return sc(i_ids, item2ent)


def _row_gather(ids, table):
    @functools.partial(
        pl.kernel,
        out_type=jax.ShapeDtypeStruct((B, D), jnp.float32),
        mesh=plsc.VectorSubcoreMesh(**_MESH),
        scratch_types=[
            pltpu.VMEM((BPW,), jnp.int32),
            pltpu.VMEM((HB, D), jnp.float32),
            pltpu.SemaphoreType.DMA,
        ],
    )
    def sc(ids_h, table_h, rows_out, idx, buf, sem):
        base = _worker_base()
        pltpu.sync_copy(ids_h.at[pl.ds(base, BPW)], idx)

        def drain_group():
            for _ in range(G):
                pltpu.make_async_copy(table_h.at[pl.ds(0, 1)],
                                      buf.at[pl.ds(0, 1)], sem).wait()

        for h in range(2):

            def fetch_group(it, carry, h=h):
                pos = h * HB + it * G
                vec = idx[pl.ds(pos, G)]
                for g in range(G):
                    r = vec[g]
                    pltpu.async_copy(table_h.at[pl.ds(r, 1)],
                                     buf.at[pl.ds(it * G + g, 1)], sem)

                @pl.when(it != 0)
                def _():
                    drain_group()

                return carry

            lax.fori_loop(0, HB // G, fetch_group, 0)
            drain_group()
            pltpu.sync_copy(buf, rows_out.at[pl.ds(base + h * HB, HB)])

    return sc(ids, table)


def _tc_dense(u_rows, i_rows, e_rows, pref_emb, pref_norm_emb, rel_emb, norm_emb):
    BLK = 4096

    def body(u_ref, i_ref, e_ref, pref_ref, pn_ref, rel_ref, nm_ref, out_ref):
        u = u_ref[...]
        ie = i_ref[...] + e_ref[...]
        x = u + ie
        dvec = u - ie
        pr = pref_ref[...] + rel_ref[...]
        pn = pn_ref[...] + nm_ref[...]
        pre = lax.dot_general(x, pr, (((1,), (1,)), ((), ())),
                              preferred_element_type=jnp.float32) * 0.5
        r_e = lax.dot_general(pre, pr, (((1,), (0,)), ((), ())),
                              preferred_element_type=jnp.float32) * 0.5
        nrm = lax.dot_general(pre, pn, (((1,), (0,)), ((), ())),
                              preferred_element_type=jnp.float32) * 0.5
        ones = jnp.ones((D, 1), jnp.float32)
        t = lax.dot_general(dvec * nrm, ones, (((1,), (0,)), ((), ())),
                            preferred_element_type=jnp.float32)
        s = lax.dot_general(jnp.abs(dvec + r_e - t * nrm), ones,
                            (((1,), (0,)), ((), ())),
                            preferred_element_type=jnp.float32)
        out_ref[...] = s[:, 0]

    return pl.pallas_call(
        body,
        grid=(B // BLK,),
        in_specs=[
            pl.BlockSpec((BLK, D), lambda i: (i, 0)),
            pl.BlockSpec((BLK, D), lambda i: (i, 0)),
            pl.BlockSpec((BLK, D), lambda i: (i, 0)),
            pl.BlockSpec((64, 64), lambda i: (0, 0)),
            pl.BlockSpec((64, 64), lambda i: (0, 0)),
            pl.BlockSpec((64, 64), lambda i: (0, 0)),
            pl.BlockSpec((64, 64), lambda i: (0, 0)),
        ],
        out_specs=pl.BlockSpec((BLK,), lambda i: (i,)),
        out_shape=jax.ShapeDtypeStruct((B,), jnp.float32),
    )(u_rows, i_rows, e_rows, pref_emb, pref_norm_emb, rel_emb, norm_emb)


def kernel(u_ids, i_ids, item2ent, user_emb, item_emb, ent_emb,
           pref_emb, pref_norm_emb, rel_emb, norm_emb):
    e_var = _ent_lookup(i_ids, item2ent)
    e_rows = _row_gather(e_var, ent_emb)
    u_rows = _row_gather(u_ids, user_emb)
    i_rows = _row_gather(i_ids, item_emb)
    return _tc_dense(u_rows, i_rows, e_rows,
                     pref_emb, pref_norm_emb, rel_emb, norm_emb)
